# Initial kernel scaffold; baseline (speedup 1.0000x reference)
import jax
import jax.numpy as jnp
from jax.experimental import pallas as pl


def kernel(in_feat_map, act_mask, W, b):
    raise NotImplementedError("write your pallas kernel here")


# speedup vs baseline: 1.0051x; 1.0051x over previous
import jax
import jax.numpy as jnp
from jax.experimental import pallas as pl


def _body(x_ref, m_ref, w_ref, b_ref, o_ref):
    xb = x_ref[0]
    mb = m_ref[0]
    proc = jax.lax.dot_general(
        w_ref[...], xb, (((0,), (0,)), ((), ())),
        preferred_element_type=jnp.float32,
    )
    proc = proc + b_ref[...]
    o_ref[0] = jnp.where(mb != 0.0, proc, xb)


def kernel(in_feat_map, act_mask, W, b):
    B, C, H, Wd = in_feat_map.shape
    HW = H * Wd
    Tp = 2048
    x = in_feat_map.reshape(B, C, HW)
    m = act_mask.astype(jnp.float32).reshape(B, 1, HW)
    out = pl.pallas_call(
        _body,
        grid=(B, HW // Tp),
        in_specs=[
            pl.BlockSpec((1, C, Tp), lambda i, j: (i, 0, j)),
            pl.BlockSpec((1, 1, Tp), lambda i, j: (i, 0, j)),
            pl.BlockSpec((C, C), lambda i, j: (0, 0)),
            pl.BlockSpec((C, 1), lambda i, j: (0, 0)),
        ],
        out_specs=pl.BlockSpec((1, C, Tp), lambda i, j: (i, 0, j)),
        out_shape=jax.ShapeDtypeStruct((B, C, HW), jnp.float32),
    )(x, m, W, b.reshape(C, 1))
    return out.reshape(B, C, H, Wd)


# speedup vs baseline: 1.0088x; 1.0037x over previous
import jax
import jax.numpy as jnp
from jax.experimental import pallas as pl


def _body(x_ref, m_ref, w_ref, b_ref, o_ref):
    xb = x_ref[0]
    mb = m_ref[0]
    proc = jax.lax.dot_general(
        w_ref[...].astype(jnp.bfloat16), xb.astype(jnp.bfloat16),
        (((0,), (0,)), ((), ())),
        preferred_element_type=jnp.float32,
    )
    proc = proc + b_ref[...]
    o_ref[0] = jnp.where(mb != 0.0, proc, xb)


def kernel(in_feat_map, act_mask, W, b):
    B, C, H, Wd = in_feat_map.shape
    HW = H * Wd
    Tp = 2048
    x = in_feat_map.reshape(B, C, HW)
    m = act_mask.astype(jnp.float32).reshape(B, 1, HW)
    out = pl.pallas_call(
        _body,
        grid=(B, HW // Tp),
        in_specs=[
            pl.BlockSpec((1, C, Tp), lambda i, j: (i, 0, j)),
            pl.BlockSpec((1, 1, Tp), lambda i, j: (i, 0, j)),
            pl.BlockSpec((C, C), lambda i, j: (0, 0)),
            pl.BlockSpec((C, 1), lambda i, j: (0, 0)),
        ],
        out_specs=pl.BlockSpec((1, C, Tp), lambda i, j: (i, 0, j)),
        out_shape=jax.ShapeDtypeStruct((B, C, HW), jnp.float32),
    )(x, m, W, b.reshape(C, 1))
    return out.reshape(B, C, H, Wd)


# speedup vs baseline: 2.9412x; 2.9157x over previous
import jax
import jax.numpy as jnp
from jax.experimental import pallas as pl


def _body(x_ref, m_ref, w_ref, b_ref, o_ref):
    C = x_ref.shape[1]
    xb = x_ref[0].reshape(C, -1)
    mb = m_ref[0].reshape(1, -1)
    proc = jax.lax.dot_general(
        w_ref[...], xb, (((0,), (0,)), ((), ())),
        preferred_element_type=jnp.float32,
    )
    proc = proc + b_ref[...]
    out = jnp.where(mb != 0.0, proc, xb)
    o_ref[0] = out.reshape(x_ref.shape[1:])


def kernel(in_feat_map, act_mask, W, b):
    B, C, H, Wd = in_feat_map.shape
    Hb = 16
    m = act_mask.astype(jnp.float32)
    out = pl.pallas_call(
        _body,
        grid=(B, H // Hb),
        in_specs=[
            pl.BlockSpec((1, C, Hb, Wd), lambda i, j: (i, 0, j, 0)),
            pl.BlockSpec((1, 1, Hb, Wd), lambda i, j: (i, 0, j, 0)),
            pl.BlockSpec((C, C), lambda i, j: (0, 0)),
            pl.BlockSpec((C, 1), lambda i, j: (0, 0)),
        ],
        out_specs=pl.BlockSpec((1, C, Hb, Wd), lambda i, j: (i, 0, j, 0)),
        out_shape=jax.ShapeDtypeStruct((B, C, H, Wd), jnp.float32),
    )(in_feat_map, m, W, b.reshape(C, 1))
    return out


# speedup vs baseline: 2.9652x; 1.0082x over previous
import jax
import jax.numpy as jnp
from jax.experimental import pallas as pl


def _body(x_ref, m_ref, w_ref, b_ref, o_ref):
    C = x_ref.shape[1]
    xb4 = x_ref[0]
    xb = xb4.reshape(C, -1)
    proc = jax.lax.dot_general(
        w_ref[...], xb, (((0,), (0,)), ((), ())),
        preferred_element_type=jnp.float32,
    )
    proc = proc + b_ref[...]
    proc4 = proc.reshape(xb4.shape)
    o_ref[0] = jnp.where(m_ref[0], proc4, xb4)


def kernel(in_feat_map, act_mask, W, b):
    B, C, H, Wd = in_feat_map.shape
    Hb = 16
    out = pl.pallas_call(
        _body,
        grid=(B, H // Hb),
        in_specs=[
            pl.BlockSpec((1, C, Hb, Wd), lambda i, j: (i, 0, j, 0)),
            pl.BlockSpec((1, 1, Hb, Wd), lambda i, j: (i, 0, j, 0)),
            pl.BlockSpec((C, C), lambda i, j: (0, 0)),
            pl.BlockSpec((C, 1), lambda i, j: (0, 0)),
        ],
        out_specs=pl.BlockSpec((1, C, Hb, Wd), lambda i, j: (i, 0, j, 0)),
        out_shape=jax.ShapeDtypeStruct((B, C, H, Wd), jnp.float32),
    )(in_feat_map, act_mask, W, b.reshape(C, 1))
    return out


# speedup vs baseline: 3.3151x; 1.1180x over previous
import jax
import jax.numpy as jnp
from jax.experimental import pallas as pl


def _body(x_ref, m_ref, w_ref, b_ref, o_ref):
    C = x_ref.shape[1]
    xb4 = x_ref[0]
    xb = xb4.reshape(C, -1)
    proc = jax.lax.dot_general(
        w_ref[...], xb, (((0,), (0,)), ((), ())),
        preferred_element_type=jnp.float32,
    )
    proc = proc + b_ref[...]
    proc4 = proc.reshape(xb4.shape)
    o_ref[0] = jnp.where(m_ref[0], proc4, xb4)


def kernel(in_feat_map, act_mask, W, b):
    B, C, H, Wd = in_feat_map.shape
    Hb = 32
    out = pl.pallas_call(
        _body,
        grid=(B, H // Hb),
        in_specs=[
            pl.BlockSpec((1, C, Hb, Wd), lambda i, j: (i, 0, j, 0)),
            pl.BlockSpec((1, 1, Hb, Wd), lambda i, j: (i, 0, j, 0)),
            pl.BlockSpec((C, C), lambda i, j: (0, 0)),
            pl.BlockSpec((C, 1), lambda i, j: (0, 0)),
        ],
        out_specs=pl.BlockSpec((1, C, Hb, Wd), lambda i, j: (i, 0, j, 0)),
        out_shape=jax.ShapeDtypeStruct((B, C, H, Wd), jnp.float32),
    )(in_feat_map, act_mask, W, b.reshape(C, 1))
    return out


# speedup vs baseline: 3.4502x; 1.0407x over previous
import jax
import jax.numpy as jnp
from jax.experimental import pallas as pl


def _body(x_ref, m_ref, w_ref, b_ref, o_ref):
    C = x_ref.shape[1]
    xb4 = x_ref[0]
    xb = xb4.reshape(C, -1)
    proc = jax.lax.dot_general(
        w_ref[...], xb, (((0,), (0,)), ((), ())),
        preferred_element_type=jnp.float32,
    )
    proc = proc + b_ref[...]
    proc4 = proc.reshape(xb4.shape)
    o_ref[0] = jnp.where(m_ref[0], proc4, xb4)


def kernel(in_feat_map, act_mask, W, b):
    B, C, H, Wd = in_feat_map.shape
    Hb = 64
    out = pl.pallas_call(
        _body,
        grid=(B, H // Hb),
        in_specs=[
            pl.BlockSpec((1, C, Hb, Wd), lambda i, j: (i, 0, j, 0)),
            pl.BlockSpec((1, 1, Hb, Wd), lambda i, j: (i, 0, j, 0)),
            pl.BlockSpec((C, C), lambda i, j: (0, 0)),
            pl.BlockSpec((C, 1), lambda i, j: (0, 0)),
        ],
        out_specs=pl.BlockSpec((1, C, Hb, Wd), lambda i, j: (i, 0, j, 0)),
        out_shape=jax.ShapeDtypeStruct((B, C, H, Wd), jnp.float32),
    )(in_feat_map, act_mask, W, b.reshape(C, 1))
    return out


# speedup vs baseline: 3.5264x; 1.0221x over previous
import jax
import jax.numpy as jnp
from jax.experimental import pallas as pl


def _body(x_ref, m_ref, w_ref, b_ref, o_ref):
    C = x_ref.shape[1]
    xb4 = x_ref[0]
    xb = xb4.astype(jnp.bfloat16).reshape(C, -1)
    proc = jax.lax.dot_general(
        w_ref[...], xb, (((0,), (0,)), ((), ())),
        preferred_element_type=jnp.float32,
    )
    proc = (proc + b_ref[...]).astype(jnp.bfloat16)
    proc4 = proc.reshape(xb4.shape).astype(jnp.float32)
    o_ref[0] = jnp.where(m_ref[0], proc4, xb4)


def kernel(in_feat_map, act_mask, W, b):
    B, C, H, Wd = in_feat_map.shape
    Hb = 64
    out = pl.pallas_call(
        _body,
        grid=(B, H // Hb),
        in_specs=[
            pl.BlockSpec((1, C, Hb, Wd), lambda i, j: (i, 0, j, 0)),
            pl.BlockSpec((1, 1, Hb, Wd), lambda i, j: (i, 0, j, 0)),
            pl.BlockSpec((C, C), lambda i, j: (0, 0)),
            pl.BlockSpec((C, 1), lambda i, j: (0, 0)),
        ],
        out_specs=pl.BlockSpec((1, C, Hb, Wd), lambda i, j: (i, 0, j, 0)),
        out_shape=jax.ShapeDtypeStruct((B, C, H, Wd), jnp.float32),
    )(in_feat_map, act_mask, W.astype(jnp.bfloat16), b.reshape(C, 1))
    return out


# speedup vs baseline: 3.5464x; 1.0057x over previous
import jax
import jax.numpy as jnp
from jax.experimental import pallas as pl
from jax.experimental.pallas import tpu as pltpu


def _body(x_ref, m_ref, w_ref, b_ref, o_ref):
    C = x_ref.shape[1]
    xb4 = x_ref[0]
    xb = xb4.astype(jnp.bfloat16).reshape(C, -1)
    proc = jax.lax.dot_general(
        w_ref[...], xb, (((0,), (0,)), ((), ())),
        preferred_element_type=jnp.float32,
    )
    proc = (proc + b_ref[...]).astype(jnp.bfloat16)
    proc4 = proc.reshape(xb4.shape).astype(jnp.float32)
    o_ref[0] = jnp.where(m_ref[0], proc4, xb4)


def kernel(in_feat_map, act_mask, W, b):
    B, C, H, Wd = in_feat_map.shape
    Hb = 64
    out = pl.pallas_call(
        _body,
        grid=(B, H // Hb),
        in_specs=[
            pl.BlockSpec((1, C, Hb, Wd), lambda i, j: (i, 0, j, 0)),
            pl.BlockSpec((1, 1, Hb, Wd), lambda i, j: (i, 0, j, 0)),
            pl.BlockSpec((C, C), lambda i, j: (0, 0)),
            pl.BlockSpec((C, 1), lambda i, j: (0, 0)),
        ],
        out_specs=pl.BlockSpec((1, C, Hb, Wd), lambda i, j: (i, 0, j, 0)),
        out_shape=jax.ShapeDtypeStruct((B, C, H, Wd), jnp.float32),
        compiler_params=pltpu.CompilerParams(
            dimension_semantics=("parallel", "parallel")),
    )(in_feat_map, act_mask, W.astype(jnp.bfloat16), b.reshape(C, 1))
    return out
